# baseline (speedup 1.0000x reference)
import functools
import jax
import jax.numpy as jnp
from jax import lax
from jax.experimental import pallas as pl
from jax.experimental.pallas import tpu as pltpu
from jax.experimental.pallas import tpu_sc as plsc

B, L, D = 1024, 200, 256
N = B * L
H = D // 2
NC, NS = 2, 16
NW = NC * NS
TPW = N // NW
CHUNK = 64
NCHUNKS = TPW // CHUNK


def _prep_body(st_ref, sw_ref, sb_ref, ow_ref, ob_ref, pe_ref,
               fused_ref, w2_ref, pel_ref):
    w1 = ow_ref[pl.ds(0, H), :]
    w2 = ow_ref[pl.ds(H, H), :]
    fused_ref[...] = jnp.dot(st_ref[...], w1, preferred_element_type=jnp.float32)
    w2_ref[...] = jnp.dot(sw_ref[...], w2, preferred_element_type=jnp.float32) * 10.0
    cvec = jnp.dot(sb_ref[...].reshape(1, H), w2, preferred_element_type=jnp.float32)
    pel_ref[...] = pe_ref[...] + cvec + ob_ref[...].reshape(1, D)


_prep = pl.pallas_call(
    _prep_body,
    out_shape=[
        jax.ShapeDtypeStruct((1000, D), jnp.float32),
        jax.ShapeDtypeStruct((1, D), jnp.float32),
        jax.ShapeDtypeStruct((L, D), jnp.float32),
    ],
)

def _sc_body(idx_ref, val_ref, tab_ref, w2_ref, pe_ref, out_ref,
             idx_all, val_all, rows, outbuf, pe_v, w2_v, gsems, ssems):
    wid = lax.axis_index("s") * NC + lax.axis_index("c")
    gbase = wid * TPW
    pltpu.sync_copy(w2_ref, w2_v)
    pltpu.sync_copy(pe_ref, pe_v)
    pltpu.sync_copy(idx_ref.at[pl.ds(gbase, TPW)], idx_all)
    pltpu.sync_copy(val_ref.at[pl.ds(gbase, TPW)], val_all)

    w2regs = [w2_v[pl.ds(16 * k, 16)] for k in range(D // 16)]

    def prefetch(u, slot):
        pltpu.async_copy(
            tab_ref.at[idx_all.at[pl.ds(u * CHUNK, CHUNK)]],
            rows.at[slot], gsems.at[slot])

    prefetch(0, 0)

    def chunk_body(t, lstart):
        p = lax.rem(t, 2)
        q = 1 - p
        base = gbase + t * CHUNK

        @pl.when(t + 1 < NCHUNKS)
        def _():
            prefetch(t + 1, q)

        @pl.when(t >= 1)
        def _():
            pltpu.make_async_copy(
                outbuf.at[q], out_ref.at[pl.ds(0, CHUNK)], ssems.at[q]).wait()

        pltpu.make_async_copy(
            tab_ref.at[idx_all.at[pl.ds(t * CHUNK, CHUNK)]],
            rows.at[p], gsems.at[p]).wait()

        vbase = t * CHUNK

        @plsc.parallel_loop(0, CHUNK, unroll=4)
        def tok_body(c):
            lraw = lstart + c
            lpos = lax.select(lraw >= L, lraw - L, lraw)
            valv = plsc.load_gather(val_all, [jnp.full((16,), vbase + c)])
            for k in range(D // 32):
                w = plsc.bitcast(rows[p, c, pl.ds(16 * k, 16)], jnp.bfloat16)
                a, b = plsc.unpack(w, format=plsc.PackFormat.INTERLEAVED)
                sl0 = pl.ds(32 * k, 16)
                sl1 = pl.ds(32 * k + 16, 16)
                outbuf[p, c, sl0] = a + valv * w2regs[2 * k] + pe_v[lpos, sl0]
                outbuf[p, c, sl1] = (b + valv * w2regs[2 * k + 1]
                                     + pe_v[lpos, sl1])
        pltpu.async_copy(outbuf.at[p], out_ref.at[pl.ds(base, CHUNK)],
                         ssems.at[p])
        lnext = lstart + CHUNK
        return lax.select(lnext >= L, lnext - L, lnext)

    lax.fori_loop(0, NCHUNKS, chunk_body, 0)
    pltpu.make_async_copy(
        outbuf.at[1], out_ref.at[pl.ds(0, CHUNK)], ssems.at[1]).wait()


_sc_call = pl.kernel(
    _sc_body,
    out_type=jax.ShapeDtypeStruct((N, D), jnp.float32),
    mesh=plsc.VectorSubcoreMesh(core_axis_name="c", subcore_axis_name="s"),
    compiler_params=pltpu.CompilerParams(needs_layout_passes=False),
    scratch_types=[
        pltpu.VMEM((TPW,), jnp.int32),
        pltpu.VMEM((TPW,), jnp.float32),
        pltpu.VMEM((2, CHUNK, H), jnp.float32),
        pltpu.VMEM((2, CHUNK, D), jnp.float32),
        pltpu.VMEM((L, D), jnp.float32),
        pltpu.VMEM((D,), jnp.float32),
        pltpu.SemaphoreType.DMA((2,)),
        pltpu.SemaphoreType.DMA((2,)),
    ],
)


@jax.jit
def kernel(x, station_table, size_W, size_b, out_W, out_b, pe):
    fused, w2, pel = _prep(station_table, size_W, size_b, out_W, out_b, pe[:L])
    tab_pk = jax.lax.bitcast_convert_type(
        fused.astype(jnp.bfloat16).reshape(1000, D // 32, 2, 16)
        .swapaxes(2, 3), jnp.float32).reshape(1000, H)
    idx = x[:, :, 0].astype(jnp.int32).reshape(-1)
    val = x[:, :, 1].reshape(-1)
    out = _sc_call(idx, val, tab_pk, w2.reshape(-1), pel)
    return out.reshape(B, L, D)


# speedup vs baseline: 1.0146x; 1.0146x over previous
import functools
import jax
import jax.numpy as jnp
from jax import lax
from jax.experimental import pallas as pl
from jax.experimental.pallas import tpu as pltpu
from jax.experimental.pallas import tpu_sc as plsc

B, L, D = 1024, 200, 256
N = B * L
H = D // 2
NC, NS = 2, 16
NW = NC * NS
TPW = N // NW
CHUNK = 64
NCHUNKS = TPW // CHUNK


def _prep_body(st_ref, sw_ref, sb_ref, ow_ref, ob_ref, pe_ref,
               fused_ref, w2_ref, pel_ref):
    w1 = ow_ref[pl.ds(0, H), :]
    w2 = ow_ref[pl.ds(H, H), :]
    fused_ref[...] = jnp.dot(st_ref[...], w1, preferred_element_type=jnp.float32)
    w2_ref[...] = jnp.dot(sw_ref[...], w2, preferred_element_type=jnp.float32) * 10.0
    cvec = jnp.dot(sb_ref[...].reshape(1, H), w2, preferred_element_type=jnp.float32)
    pel_ref[...] = pe_ref[...] + cvec + ob_ref[...].reshape(1, D)


_prep = pl.pallas_call(
    _prep_body,
    out_shape=[
        jax.ShapeDtypeStruct((1000, D), jnp.float32),
        jax.ShapeDtypeStruct((1, D), jnp.float32),
        jax.ShapeDtypeStruct((L, D), jnp.float32),
    ],
)

def _sc_body(idx_ref, val_ref, tab_ref, w2_ref, pe_ref, out_ref,
             idx_all, val_all, rows, outbuf, pe_v, w2_v, gsems, ssems):
    wid = lax.axis_index("s") * NC + lax.axis_index("c")
    gbase = wid * TPW
    pltpu.sync_copy(w2_ref, w2_v)
    pltpu.sync_copy(pe_ref, pe_v)
    pltpu.sync_copy(idx_ref.at[pl.ds(gbase, TPW)], idx_all)
    pltpu.sync_copy(val_ref.at[pl.ds(gbase, TPW)], val_all)

    w2regs = [w2_v[pl.ds(16 * k, 16)] for k in range(D // 16)]

    def prefetch(u, slot):
        pltpu.async_copy(
            tab_ref.at[idx_all.at[pl.ds(u * CHUNK, CHUNK)]],
            rows.at[slot], gsems.at[slot])

    prefetch(0, 0)

    def chunk_body(t, lstart):
        p = lax.rem(t, 2)
        q = 1 - p
        base = gbase + t * CHUNK

        @pl.when(t + 1 < NCHUNKS)
        def _():
            prefetch(t + 1, q)

        @pl.when(t >= 1)
        def _():
            pltpu.make_async_copy(
                outbuf.at[q], out_ref.at[pl.ds(0, CHUNK)], ssems.at[q]).wait()

        pltpu.make_async_copy(
            tab_ref.at[idx_all.at[pl.ds(t * CHUNK, CHUNK)]],
            rows.at[p], gsems.at[p]).wait()

        vbase = t * CHUNK

        @plsc.parallel_loop(0, CHUNK, unroll=2)
        def tok_body(c):
            lraw = lstart + c
            lpos = lax.select(lraw >= L, lraw - L, lraw)
            valv = plsc.load_gather(val_all, [jnp.full((16,), vbase + c)])
            for k in range(D // 32):
                ui = plsc.bitcast(rows[p, c, pl.ds(16 * k, 16)], jnp.uint32)
                a = plsc.bitcast(ui << 16, jnp.float32)
                b = plsc.bitcast(ui & jnp.uint32(0xFFFF0000), jnp.float32)
                sl0 = pl.ds(32 * k, 16)
                sl1 = pl.ds(32 * k + 16, 16)
                outbuf[p, c, sl0] = a + valv * w2regs[2 * k] + pe_v[lpos, sl0]
                outbuf[p, c, sl1] = (b + valv * w2regs[2 * k + 1]
                                     + pe_v[lpos, sl1])
        pltpu.async_copy(outbuf.at[p], out_ref.at[pl.ds(base, CHUNK)],
                         ssems.at[p])
        lnext = lstart + CHUNK
        return lax.select(lnext >= L, lnext - L, lnext)

    lax.fori_loop(0, NCHUNKS, chunk_body, 0)
    pltpu.make_async_copy(
        outbuf.at[1], out_ref.at[pl.ds(0, CHUNK)], ssems.at[1]).wait()


_sc_call = pl.kernel(
    _sc_body,
    out_type=jax.ShapeDtypeStruct((N, D), jnp.float32),
    mesh=plsc.VectorSubcoreMesh(core_axis_name="c", subcore_axis_name="s"),
    compiler_params=pltpu.CompilerParams(needs_layout_passes=False),
    scratch_types=[
        pltpu.VMEM((TPW,), jnp.int32),
        pltpu.VMEM((TPW,), jnp.float32),
        pltpu.VMEM((2, CHUNK, H), jnp.float32),
        pltpu.VMEM((2, CHUNK, D), jnp.float32),
        pltpu.VMEM((L, D), jnp.float32),
        pltpu.VMEM((D,), jnp.float32),
        pltpu.SemaphoreType.DMA((2,)),
        pltpu.SemaphoreType.DMA((2,)),
    ],
)


@jax.jit
def kernel(x, station_table, size_W, size_b, out_W, out_b, pe):
    fused, w2, pel = _prep(station_table, size_W, size_b, out_W, out_b, pe[:L])
    tab_pk = jax.lax.bitcast_convert_type(
        fused.astype(jnp.bfloat16).reshape(1000, D // 32, 2, 16)
        .swapaxes(2, 3), jnp.float32).reshape(1000, H)
    idx = x[:, :, 0].astype(jnp.int32).reshape(-1)
    val = x[:, :, 1].reshape(-1)
    out = _sc_call(idx, val, tab_pk, w2.reshape(-1), pel)
    return out.reshape(B, L, D)


# speedup vs baseline: 1.2178x; 1.2003x over previous
import functools
import jax
import jax.numpy as jnp
from jax import lax
from jax.experimental import pallas as pl
from jax.experimental.pallas import tpu as pltpu
from jax.experimental.pallas import tpu_sc as plsc

B, L, D = 1024, 200, 256
N = B * L
H = D // 2
NC, NS = 2, 16
NW = NC * NS
TPW = N // NW
CHUNK = 64
NCHUNKS = TPW // CHUNK


def _prep_body(st_ref, sw_ref, sb_ref, ow_ref, ob_ref, pe_ref,
               fused_ref, w2_ref, pel_ref):
    w1 = ow_ref[pl.ds(0, H), :]
    w2 = ow_ref[pl.ds(H, H), :]
    fused_ref[...] = jnp.dot(st_ref[...], w1, preferred_element_type=jnp.float32)
    w2_ref[...] = jnp.dot(sw_ref[...], w2, preferred_element_type=jnp.float32) * 10.0
    cvec = jnp.dot(sb_ref[...].reshape(1, H), w2, preferred_element_type=jnp.float32)
    pel_ref[...] = pe_ref[...] + cvec + ob_ref[...].reshape(1, D)


_prep = pl.pallas_call(
    _prep_body,
    out_shape=[
        jax.ShapeDtypeStruct((1000, D), jnp.float32),
        jax.ShapeDtypeStruct((1, D), jnp.float32),
        jax.ShapeDtypeStruct((L, D), jnp.float32),
    ],
)

def _sc_body(idx_ref, val_ref, tab_ref, w2_ref, pe_ref, out_ref,
             idx_all, val_all, rows, outbuf, pe_v, w2_v, gsems, ssems):
    wid = lax.axis_index("s") * NC + lax.axis_index("c")
    gbase = wid * TPW
    pltpu.sync_copy(w2_ref, w2_v)
    pltpu.sync_copy(pe_ref, pe_v)
    pltpu.sync_copy(idx_ref.at[pl.ds(gbase, TPW)], idx_all)
    pltpu.sync_copy(val_ref.at[pl.ds(gbase, TPW)], val_all)

    w2regs = [w2_v[pl.ds(16 * k, 16)] for k in range(D // 16)]

    def prefetch(u, slot):
        pltpu.async_copy(
            tab_ref.at[idx_all.at[pl.ds(u * CHUNK, CHUNK)]],
            rows.at[slot], gsems.at[slot])

    prefetch(0, 0)
    prefetch(1, 1)

    def chunk_body(t, lstart):
        p = lax.rem(t, 4)
        o = lax.rem(t, 2)
        base = gbase + t * CHUNK

        @pl.when(t + 2 < NCHUNKS)
        def _():
            prefetch(t + 2, lax.rem(t + 2, 4))

        pltpu.make_async_copy(
            tab_ref.at[idx_all.at[pl.ds(t * CHUNK, CHUNK)]],
            rows.at[p], gsems.at[p]).wait()

        @pl.when(t >= 2)
        def _():
            pltpu.make_async_copy(
                outbuf.at[o], out_ref.at[pl.ds(0, CHUNK)], ssems.at[o]).wait()

        vbase = t * CHUNK

        @plsc.parallel_loop(0, CHUNK, unroll=2)
        def tok_body(c):
            lraw = lstart + c
            lpos = lax.select(lraw >= L, lraw - L, lraw)
            valv = plsc.load_gather(val_all, [jnp.full((16,), vbase + c)])
            for k in range(D // 32):
                ui = plsc.bitcast(rows[p, c, pl.ds(16 * k, 16)], jnp.uint32)
                a = plsc.bitcast(ui << 16, jnp.float32)
                b = plsc.bitcast(ui & jnp.uint32(0xFFFF0000), jnp.float32)
                sl0 = pl.ds(32 * k, 16)
                sl1 = pl.ds(32 * k + 16, 16)
                outbuf[o, c, sl0] = a + valv * w2regs[2 * k] + pe_v[lpos, sl0]
                outbuf[o, c, sl1] = (b + valv * w2regs[2 * k + 1]
                                     + pe_v[lpos, sl1])
        pltpu.async_copy(outbuf.at[o], out_ref.at[pl.ds(base, CHUNK)],
                         ssems.at[o])
        lnext = lstart + CHUNK
        return lax.select(lnext >= L, lnext - L, lnext)

    lax.fori_loop(0, NCHUNKS, chunk_body, 0)
    pltpu.make_async_copy(
        outbuf.at[0], out_ref.at[pl.ds(0, CHUNK)], ssems.at[0]).wait()
    pltpu.make_async_copy(
        outbuf.at[1], out_ref.at[pl.ds(0, CHUNK)], ssems.at[1]).wait()


_sc_call = pl.kernel(
    _sc_body,
    out_type=jax.ShapeDtypeStruct((N, D), jnp.float32),
    mesh=plsc.VectorSubcoreMesh(core_axis_name="c", subcore_axis_name="s"),
    compiler_params=pltpu.CompilerParams(needs_layout_passes=False),
    scratch_types=[
        pltpu.VMEM((TPW,), jnp.int32),
        pltpu.VMEM((TPW,), jnp.float32),
        pltpu.VMEM((4, CHUNK, H), jnp.float32),
        pltpu.VMEM((2, CHUNK, D), jnp.float32),
        pltpu.VMEM((L, D), jnp.float32),
        pltpu.VMEM((D,), jnp.float32),
        pltpu.SemaphoreType.DMA((4,)),
        pltpu.SemaphoreType.DMA((2,)),
    ],
)


@jax.jit
def kernel(x, station_table, size_W, size_b, out_W, out_b, pe):
    fused, w2, pel = _prep(station_table, size_W, size_b, out_W, out_b, pe[:L])
    tab_pk = jax.lax.bitcast_convert_type(
        fused.astype(jnp.bfloat16).reshape(1000, D // 32, 2, 16)
        .swapaxes(2, 3), jnp.float32).reshape(1000, H)
    idx = x[:, :, 0].astype(jnp.int32).reshape(-1)
    val = x[:, :, 1].reshape(-1)
    out = _sc_call(idx, val, tab_pk, w2.reshape(-1), pel)
    return out.reshape(B, L, D)
